# Initial kernel scaffold; baseline (speedup 1.0000x reference)
import functools

import jax
import jax.numpy as jnp
from jax import lax
from jax.experimental import pallas as pl
from jax.experimental.pallas import tpu as pltpu
from jax.experimental.pallas import tpu_sc as plsc

_NC = 2
_NS = 16
_NW = _NC * _NS
_L = 16
_CH = 128

_LN2 = 0.6931471805599453
_SQRT2 = 1.4142135381698608


def _ln(y):
    yi = lax.bitcast_convert_type(y, jnp.int32)
    ex = lax.shift_right_arithmetic(yi, 23) - 127
    mi = lax.bitwise_or(lax.bitwise_and(yi, 0x007FFFFF), 0x3F800000)
    m = lax.bitcast_convert_type(mi, jnp.float32)
    big = m >= _SQRT2
    m = jnp.where(big, m * jnp.float32(0.5), m)
    e = ex.astype(jnp.float32) + jnp.where(big, jnp.float32(1.0), jnp.float32(0.0))
    f = m - jnp.float32(1.0)
    q = jnp.float32(1.0 / 9.0)
    for c in (-1.0 / 8, 1.0 / 7, -1.0 / 6, 1.0 / 5, -1.0 / 4, 1.0 / 3,
              -1.0 / 2, 1.0):
        q = q * f + jnp.float32(c)
    return e * jnp.float32(_LN2) + f * q


def _sc_body(nch, d, idx_hbm, table_hbm, out_hbm, idx_v, rows_v, out_v, sem):
    cid = lax.axis_index("c")
    sid = lax.axis_index("s")
    wid = sid * _NC + cid
    kc = d // _L

    def chunk_body(c, carry):
        pltpu.sync_copy(idx_hbm.at[wid, c], idx_v)
        cps = [
            pltpu.async_copy(table_hbm.at[idx_v.at[j]], rows_v.at[j], sem)
            for j in range(3)
        ]
        for cp in cps:
            cp.wait()

        lanes = lax.iota(jnp.int32, _L)

        def group_body(g, carry2):
            w2 = jnp.zeros((_L,), jnp.float32)
            l2 = jnp.zeros((_L,), jnp.float32)
            for tt in range(_L):
                t = g * _L + tt
                aw = jnp.zeros((_L,), jnp.float32)
                al = jnp.zeros((_L,), jnp.float32)
                for k in range(kc):
                    hv = rows_v[0, t, pl.ds(k * _L, _L)]
                    wv = rows_v[1, t, pl.ds(k * _L, _L)]
                    lv = rows_v[2, t, pl.ds(k * _L, _L)]
                    dw = hv - wv
                    aw = aw + dw * dw
                    dl = hv - lv
                    al = al + dl * dl
                msk = lanes == tt
                w2 = jnp.where(msk, jnp.sum(aw), w2)
                l2 = jnp.where(msk, jnp.sum(al), l2)
            y = jnp.float32(1.0) + jnp.exp(w2 - l2)
            out_v[pl.ds(c * _CH + g * _L, _L)] = _ln(y)
            return carry2

        lax.fori_loop(0, _CH // _L, group_body, 0)
        return carry

    lax.fori_loop(0, nch, chunk_body, 0)
    pltpu.sync_copy(out_v, out_hbm.at[pl.ds(wid * (nch * _CH), nch * _CH)])


def kernel(h_w_l, embedding):
    b = h_w_l.shape[0]
    n, d = embedding.shape
    bpw = b // _NW
    nch = bpw // _CH
    idx_all = h_w_l.reshape(_NW, nch, _CH, 3).transpose(0, 1, 3, 2)

    mesh = plsc.VectorSubcoreMesh(core_axis_name="c", subcore_axis_name="s")
    fn = pl.kernel(
        functools.partial(_sc_body, nch, d),
        out_type=jax.ShapeDtypeStruct((b,), jnp.float32),
        mesh=mesh,
        scratch_types=[
            pltpu.VMEM((3, _CH), jnp.int32),
            pltpu.VMEM((3, _CH, d), jnp.float32),
            pltpu.VMEM((bpw,), jnp.float32),
            pltpu.SemaphoreType.DMA,
        ],
    )
    return fn(idx_all, embedding)


# speedup vs baseline: 1.5803x; 1.5803x over previous
import functools

import jax
import jax.numpy as jnp
from jax import lax
from jax.experimental import pallas as pl
from jax.experimental.pallas import tpu as pltpu
from jax.experimental.pallas import tpu_sc as plsc

_NC = 2
_NS = 16
_NW = _NC * _NS
_L = 16
_CH = 128

_LN2 = 0.6931471805599453
_SQRT2 = 1.4142135381698608


def _ln(y):
    yi = lax.bitcast_convert_type(y, jnp.int32)
    ex = lax.shift_right_arithmetic(yi, 23) - 127
    mi = lax.bitwise_or(lax.bitwise_and(yi, 0x007FFFFF), 0x3F800000)
    m = lax.bitcast_convert_type(mi, jnp.float32)
    big = m >= _SQRT2
    m = jnp.where(big, m * jnp.float32(0.5), m)
    e = ex.astype(jnp.float32) + jnp.where(big, jnp.float32(1.0), jnp.float32(0.0))
    f = m - jnp.float32(1.0)
    q = jnp.float32(1.0 / 9.0)
    for c in (-1.0 / 8, 1.0 / 7, -1.0 / 6, 1.0 / 5, -1.0 / 4, 1.0 / 3,
              -1.0 / 2, 1.0):
        q = q * f + jnp.float32(c)
    return e * jnp.float32(_LN2) + f * q


def _sc_body(nch, d, idx_hbm, table_hbm, out_hbm, idx_v, rows_v, out_v, sem):
    cid = lax.axis_index("c")
    sid = lax.axis_index("s")
    wid = sid * _NC + cid
    kc = d // _L

    def chunk_body(c, carry):
        pltpu.sync_copy(idx_hbm.at[wid, c], idx_v)
        cps = [
            pltpu.async_copy(table_hbm.at[idx_v.at[j]], rows_v.at[j], sem)
            for j in range(3)
        ]
        for cp in cps:
            cp.wait()

        lanes = lax.iota(jnp.int32, _L)

        def group_body(g, carry2):
            w2 = jnp.zeros((_L,), jnp.float32)
            l2 = jnp.zeros((_L,), jnp.float32)
            for tt in range(_L):
                t = g * _L + tt
                aw = jnp.zeros((_L,), jnp.float32)
                al = jnp.zeros((_L,), jnp.float32)
                for k in range(kc):
                    hv = rows_v[0, t, pl.ds(k * _L, _L)]
                    wv = rows_v[1, t, pl.ds(k * _L, _L)]
                    lv = rows_v[2, t, pl.ds(k * _L, _L)]
                    dw = hv - wv
                    aw = aw + dw * dw
                    dl = hv - lv
                    al = al + dl * dl
                msk = lanes == tt
                w2 = jnp.where(msk, jnp.sum(aw), w2)
                l2 = jnp.where(msk, jnp.sum(al), l2)
            y = jnp.float32(1.0) + jnp.exp(w2 - l2)
            out_v[pl.ds(c * _CH + g * _L, _L)] = _ln(y)
            return carry2

        lax.fori_loop(0, _CH // _L, group_body, 0)
        return carry

    lax.fori_loop(0, nch, chunk_body, 0)
    pltpu.sync_copy(out_v, out_hbm.at[pl.ds(wid * (nch * _CH), nch * _CH)])


def kernel(h_w_l, embedding):
    b = h_w_l.shape[0]
    n, d = embedding.shape
    bpw = b // _NW
    nch = bpw // _CH
    idx_all = h_w_l.reshape(_NW, nch, _CH, 3).transpose(0, 1, 3, 2)

    mesh = plsc.VectorSubcoreMesh(core_axis_name="c", subcore_axis_name="s")
    fn = pl.kernel(
        functools.partial(_sc_body, nch, d),
        out_type=jax.ShapeDtypeStruct((b,), jnp.float32),
        mesh=mesh,
        compiler_params=pltpu.CompilerParams(needs_layout_passes=False),
        scratch_types=[
            pltpu.VMEM((3, _CH), jnp.int32),
            pltpu.VMEM((3, _CH, d), jnp.float32),
            pltpu.VMEM((bpw,), jnp.float32),
            pltpu.SemaphoreType.DMA,
        ],
    )
    return fn(idx_all, embedding)


# speedup vs baseline: 1.7446x; 1.1040x over previous
import functools

import jax
import jax.numpy as jnp
from jax import lax
from jax.experimental import pallas as pl
from jax.experimental.pallas import tpu as pltpu
from jax.experimental.pallas import tpu_sc as plsc

_NC = 2
_NS = 16
_NW = _NC * _NS
_L = 16
_CH = 128

_LN2 = 0.6931471805599453
_SQRT2 = 1.4142135381698608


def _ln(y):
    yi = lax.bitcast_convert_type(y, jnp.int32)
    ex = lax.shift_right_arithmetic(yi, 23) - 127
    mi = lax.bitwise_or(lax.bitwise_and(yi, 0x007FFFFF), 0x3F800000)
    m = lax.bitcast_convert_type(mi, jnp.float32)
    big = m >= _SQRT2
    m = jnp.where(big, m * jnp.float32(0.5), m)
    e = ex.astype(jnp.float32) + jnp.where(big, jnp.float32(1.0), jnp.float32(0.0))
    f = m - jnp.float32(1.0)
    q = jnp.float32(1.0 / 9.0)
    for c in (-1.0 / 8, 1.0 / 7, -1.0 / 6, 1.0 / 5, -1.0 / 4, 1.0 / 3,
              -1.0 / 2, 1.0):
        q = q * f + jnp.float32(c)
    return e * jnp.float32(_LN2) + f * q


def _sc_body(nch, d, idx_hbm, table_hbm, out_hbm, idx_v, rows_v, out_v, sem):
    cid = lax.axis_index("c")
    sid = lax.axis_index("s")
    wid = sid * _NC + cid
    kc = d // _L

    def fetch(c, buf):
        pltpu.sync_copy(idx_hbm.at[wid, c], idx_v.at[buf])
        for j in range(3):
            pltpu.async_copy(
                table_hbm.at[idx_v.at[buf, j]], rows_v.at[buf, j], sem.at[buf]
            )

    def drain(buf):
        for j in range(3):
            pltpu.make_async_copy(
                table_hbm.at[idx_v.at[buf, j]], rows_v.at[buf, j], sem.at[buf]
            ).wait()

    fetch(0, 0)

    def chunk_body(c, carry):
        buf = lax.rem(c, 2)
        nbuf = lax.rem(c + 1, 2)

        @pl.when(c + 1 < nch)
        def _():
            fetch(c + 1, nbuf)

        drain(buf)
        lanes = lax.iota(jnp.int32, _L)

        def group_body(g, carry2):
            w2 = jnp.zeros((_L,), jnp.float32)
            l2 = jnp.zeros((_L,), jnp.float32)
            for tt in range(_L):
                t = g * _L + tt
                aw = jnp.zeros((_L,), jnp.float32)
                al = jnp.zeros((_L,), jnp.float32)
                for k in range(kc):
                    hv = rows_v[buf, 0, t, pl.ds(k * _L, _L)]
                    wv = rows_v[buf, 1, t, pl.ds(k * _L, _L)]
                    lv = rows_v[buf, 2, t, pl.ds(k * _L, _L)]
                    dw = hv - wv
                    aw = aw + dw * dw
                    dl = hv - lv
                    al = al + dl * dl
                msk = lanes == tt
                w2 = jnp.where(msk, jnp.sum(aw), w2)
                l2 = jnp.where(msk, jnp.sum(al), l2)
            y = jnp.float32(1.0) + jnp.exp(w2 - l2)
            out_v[pl.ds(c * _CH + g * _L, _L)] = _ln(y)
            return carry2

        lax.fori_loop(0, _CH // _L, group_body, 0)
        return carry

    lax.fori_loop(0, nch, chunk_body, 0)
    pltpu.sync_copy(out_v, out_hbm.at[pl.ds(wid * (nch * _CH), nch * _CH)])


def kernel(h_w_l, embedding):
    b = h_w_l.shape[0]
    n, d = embedding.shape
    bpw = b // _NW
    nch = bpw // _CH
    idx_all = h_w_l.reshape(_NW, nch, _CH, 3).transpose(0, 1, 3, 2)

    mesh = plsc.VectorSubcoreMesh(core_axis_name="c", subcore_axis_name="s")
    fn = pl.kernel(
        functools.partial(_sc_body, nch, d),
        out_type=jax.ShapeDtypeStruct((b,), jnp.float32),
        mesh=mesh,
        compiler_params=pltpu.CompilerParams(needs_layout_passes=False),
        scratch_types=[
            pltpu.VMEM((2, 3, _CH), jnp.int32),
            pltpu.VMEM((2, 3, _CH, d), jnp.float32),
            pltpu.VMEM((bpw,), jnp.float32),
            pltpu.SemaphoreType.DMA((2,)),
        ],
    )
    return fn(idx_all, embedding)


# speedup vs baseline: 1.7949x; 1.0289x over previous
import functools

import jax
import jax.numpy as jnp
from jax import lax
from jax.experimental import pallas as pl
from jax.experimental.pallas import tpu as pltpu
from jax.experimental.pallas import tpu_sc as plsc

_NC = 2
_NS = 16
_NW = _NC * _NS
_L = 16
_CH = 128

_LN2 = 0.6931471805599453
_SQRT2 = 1.4142135381698608


def _ln(y):
    yi = lax.bitcast_convert_type(y, jnp.int32)
    ex = lax.shift_right_arithmetic(yi, 23) - 127
    mi = lax.bitwise_or(lax.bitwise_and(yi, 0x007FFFFF), 0x3F800000)
    m = lax.bitcast_convert_type(mi, jnp.float32)
    big = m >= _SQRT2
    m = jnp.where(big, m * jnp.float32(0.5), m)
    e = ex.astype(jnp.float32) + jnp.where(big, jnp.float32(1.0), jnp.float32(0.0))
    f = m - jnp.float32(1.0)
    q = jnp.float32(1.0 / 9.0)
    for c in (-1.0 / 8, 1.0 / 7, -1.0 / 6, 1.0 / 5, -1.0 / 4, 1.0 / 3,
              -1.0 / 2, 1.0):
        q = q * f + jnp.float32(c)
    return e * jnp.float32(_LN2) + f * q


def _sc_body(nch, d, idx_hbm, table_hbm, out_hbm, idx_v, rows_v, out_v, sem):
    cid = lax.axis_index("c")
    sid = lax.axis_index("s")
    wid = sid * _NC + cid
    kc = d // _L

    def fetch(c, buf):
        pltpu.sync_copy(idx_hbm.at[wid, c], idx_v.at[buf])
        for j in range(3):
            pltpu.async_copy(
                table_hbm.at[idx_v.at[buf, j]], rows_v.at[buf, j], sem.at[buf]
            )

    def drain(buf):
        for j in range(3):
            pltpu.make_async_copy(
                table_hbm.at[idx_v.at[buf, j]], rows_v.at[buf, j], sem.at[buf]
            ).wait()

    def compute_chunk(c, buf):
        drain(buf)
        lanes = lax.iota(jnp.int32, _L)

        def group_body(g, carry2):
            w2 = jnp.zeros((_L,), jnp.float32)
            l2 = jnp.zeros((_L,), jnp.float32)
            for tt in range(_L):
                t = g * _L + tt
                aw = jnp.zeros((_L,), jnp.float32)
                al = jnp.zeros((_L,), jnp.float32)
                for k in range(kc):
                    hv = rows_v[buf, 0, t, pl.ds(k * _L, _L)]
                    wv = rows_v[buf, 1, t, pl.ds(k * _L, _L)]
                    lv = rows_v[buf, 2, t, pl.ds(k * _L, _L)]
                    dw = hv - wv
                    aw = aw + dw * dw
                    dl = hv - lv
                    al = al + dl * dl
                msk = lanes == tt
                w2 = jnp.where(msk, jnp.sum(aw), w2)
                l2 = jnp.where(msk, jnp.sum(al), l2)
            y = jnp.float32(1.0) + jnp.exp(w2 - l2)
            out_v[pl.ds(c * _CH + g * _L, _L)] = _ln(y)
            return carry2

        lax.fori_loop(0, _CH // _L, group_body, 0)

    fetch(0, 0)

    def pair_body(p, carry):
        c0 = p * 2

        @pl.when(c0 + 1 < nch)
        def _():
            fetch(c0 + 1, 1)

        compute_chunk(c0, 0)

        @pl.when(c0 + 2 < nch)
        def _():
            fetch(c0 + 2, 0)

        @pl.when(c0 + 1 < nch)
        def _():
            compute_chunk(c0 + 1, 1)

        return carry

    lax.fori_loop(0, (nch + 1) // 2, pair_body, 0)
    pltpu.sync_copy(out_v, out_hbm.at[pl.ds(wid * (nch * _CH), nch * _CH)])


def kernel(h_w_l, embedding):
    b = h_w_l.shape[0]
    n, d = embedding.shape
    bpw = b // _NW
    nch = bpw // _CH
    idx_all = h_w_l.reshape(_NW, nch, _CH, 3).transpose(0, 1, 3, 2)

    mesh = plsc.VectorSubcoreMesh(core_axis_name="c", subcore_axis_name="s")
    fn = pl.kernel(
        functools.partial(_sc_body, nch, d),
        out_type=jax.ShapeDtypeStruct((b,), jnp.float32),
        mesh=mesh,
        compiler_params=pltpu.CompilerParams(needs_layout_passes=False),
        scratch_types=[
            pltpu.VMEM((2, 3, _CH), jnp.int32),
            pltpu.VMEM((2, 3, _CH, d), jnp.float32),
            pltpu.VMEM((bpw,), jnp.float32),
            pltpu.SemaphoreType.DMA((2,)),
        ],
    )
    return fn(idx_all, embedding)


# speedup vs baseline: 1.8811x; 1.0480x over previous
import functools

import jax
import jax.numpy as jnp
from jax import lax
from jax.experimental import pallas as pl
from jax.experimental.pallas import tpu as pltpu
from jax.experimental.pallas import tpu_sc as plsc

_NC = 2
_NS = 16
_NW = _NC * _NS
_L = 16
_CH = 128

_LN2 = 0.6931471805599453
_SQRT2 = 1.4142135381698608


def _ln(y):
    yi = lax.bitcast_convert_type(y, jnp.int32)
    ex = lax.shift_right_arithmetic(yi, 23) - 127
    mi = lax.bitwise_or(lax.bitwise_and(yi, 0x007FFFFF), 0x3F800000)
    m = lax.bitcast_convert_type(mi, jnp.float32)
    big = m >= _SQRT2
    m = jnp.where(big, m * jnp.float32(0.5), m)
    e = ex.astype(jnp.float32) + jnp.where(big, jnp.float32(1.0), jnp.float32(0.0))
    f = m - jnp.float32(1.0)
    q = jnp.float32(1.0 / 9.0)
    for c in (-1.0 / 8, 1.0 / 7, -1.0 / 6, 1.0 / 5, -1.0 / 4, 1.0 / 3,
              -1.0 / 2, 1.0):
        q = q * f + jnp.float32(c)
    return e * jnp.float32(_LN2) + f * q


def _sc_body(nch, d, idx_hbm, table_hbm, out_hbm, idx_v, rows_v, out_v, sem):
    cid = lax.axis_index("c")
    sid = lax.axis_index("s")
    wid = sid * _NC + cid
    kc = d // _L

    def fetch(c, buf):
        pltpu.sync_copy(idx_hbm.at[wid, c], idx_v.at[buf])
        for j in range(3):
            pltpu.async_copy(
                table_hbm.at[idx_v.at[buf, j]], rows_v.at[buf, j], sem.at[buf]
            )

    def drain(buf):
        for j in range(3):
            pltpu.make_async_copy(
                table_hbm.at[idx_v.at[buf, j]], rows_v.at[buf, j], sem.at[buf]
            ).wait()

    def compute_chunk(c, buf):
        drain(buf)
        lanes = lax.iota(jnp.int32, _L)

        def group_body(g, carry2):
            dv = jnp.zeros((_L,), jnp.float32)
            for tt in range(_L):
                t = g * _L + tt
                acc = jnp.zeros((_L,), jnp.float32)
                for k in range(kc):
                    hv = rows_v[buf, 0, t, pl.ds(k * _L, _L)]
                    wv = rows_v[buf, 1, t, pl.ds(k * _L, _L)]
                    lv = rows_v[buf, 2, t, pl.ds(k * _L, _L)]
                    hw = hv - wv
                    hl = hv - lv
                    acc = acc + (hw + hl) * (hw - hl)
                dv = jnp.where(lanes == tt, jnp.sum(acc), dv)
            y = jnp.float32(1.0) + jnp.exp(dv)
            out_v[pl.ds(c * _CH + g * _L, _L)] = _ln(y)
            return carry2

        lax.fori_loop(0, _CH // _L, group_body, 0)

    fetch(0, 0)

    def pair_body(p, carry):
        c0 = p * 2

        @pl.when(c0 + 1 < nch)
        def _():
            fetch(c0 + 1, 1)

        compute_chunk(c0, 0)

        @pl.when(c0 + 2 < nch)
        def _():
            fetch(c0 + 2, 0)

        @pl.when(c0 + 1 < nch)
        def _():
            compute_chunk(c0 + 1, 1)

        return carry

    lax.fori_loop(0, (nch + 1) // 2, pair_body, 0)
    pltpu.sync_copy(out_v, out_hbm.at[pl.ds(wid * (nch * _CH), nch * _CH)])


def kernel(h_w_l, embedding):
    b = h_w_l.shape[0]
    n, d = embedding.shape
    bpw = b // _NW
    nch = bpw // _CH
    idx_all = h_w_l.reshape(_NW, nch, _CH, 3).transpose(0, 1, 3, 2)

    mesh = plsc.VectorSubcoreMesh(core_axis_name="c", subcore_axis_name="s")
    fn = pl.kernel(
        functools.partial(_sc_body, nch, d),
        out_type=jax.ShapeDtypeStruct((b,), jnp.float32),
        mesh=mesh,
        compiler_params=pltpu.CompilerParams(needs_layout_passes=False),
        scratch_types=[
            pltpu.VMEM((2, 3, _CH), jnp.int32),
            pltpu.VMEM((2, 3, _CH, d), jnp.float32),
            pltpu.VMEM((bpw,), jnp.float32),
            pltpu.SemaphoreType.DMA((2,)),
        ],
    )
    return fn(idx_all, embedding)


# speedup vs baseline: 1.9153x; 1.0182x over previous
import functools

import jax
import jax.numpy as jnp
from jax import lax
from jax.experimental import pallas as pl
from jax.experimental.pallas import tpu as pltpu
from jax.experimental.pallas import tpu_sc as plsc

_NC = 2
_NS = 16
_NW = _NC * _NS
_L = 16
_CH = 128

_LN2 = 0.6931471805599453
_SQRT2 = 1.4142135381698608


def _ln(y):
    yi = lax.bitcast_convert_type(y, jnp.int32)
    ex = lax.shift_right_arithmetic(yi, 23) - 127
    mi = lax.bitwise_or(lax.bitwise_and(yi, 0x007FFFFF), 0x3F800000)
    m = lax.bitcast_convert_type(mi, jnp.float32)
    big = m >= _SQRT2
    m = jnp.where(big, m * jnp.float32(0.5), m)
    e = ex.astype(jnp.float32) + jnp.where(big, jnp.float32(1.0), jnp.float32(0.0))
    f = m - jnp.float32(1.0)
    q = jnp.float32(1.0 / 9.0)
    for c in (-1.0 / 8, 1.0 / 7, -1.0 / 6, 1.0 / 5, -1.0 / 4, 1.0 / 3,
              -1.0 / 2, 1.0):
        q = q * f + jnp.float32(c)
    return e * jnp.float32(_LN2) + f * q


def _sc_body(nch, d, idx_hbm, table_hbm, out_hbm, idx_v, rows_v, out_v, sem):
    cid = lax.axis_index("c")
    sid = lax.axis_index("s")
    wid = sid * _NC + cid
    kc = d // _L

    pltpu.sync_copy(idx_hbm.at[wid], idx_v)

    def fetch(c, buf):
        for j in range(3):
            pltpu.async_copy(
                table_hbm.at[idx_v.at[c, j]], rows_v.at[buf, j], sem.at[buf]
            )

    def drain(c, buf):
        for j in range(3):
            pltpu.make_async_copy(
                table_hbm.at[idx_v.at[c, j]], rows_v.at[buf, j], sem.at[buf]
            ).wait()

    def compute_chunk(c, buf):
        drain(c, buf)
        lanes = lax.iota(jnp.int32, _L)

        def group_body(g, carry2):
            dv = jnp.zeros((_L,), jnp.float32)
            for tt in range(_L):
                t = g * _L + tt
                acc = jnp.zeros((_L,), jnp.float32)
                for k in range(kc):
                    hv = rows_v[buf, 0, t, pl.ds(k * _L, _L)]
                    wv = rows_v[buf, 1, t, pl.ds(k * _L, _L)]
                    lv = rows_v[buf, 2, t, pl.ds(k * _L, _L)]
                    hw = hv - wv
                    hl = hv - lv
                    acc = acc + (hw + hl) * (hw - hl)
                dv = jnp.where(lanes == tt, jnp.sum(acc), dv)
            y = jnp.float32(1.0) + jnp.exp(dv)
            out_v[pl.ds(c * _CH + g * _L, _L)] = _ln(y)
            return carry2

        lax.fori_loop(0, _CH // _L, group_body, 0)

    fetch(0, 0)

    def pair_body(p, carry):
        c0 = p * 2

        @pl.when(c0 + 1 < nch)
        def _():
            fetch(c0 + 1, 1)

        compute_chunk(c0, 0)

        @pl.when(c0 + 2 < nch)
        def _():
            fetch(c0 + 2, 0)

        @pl.when(c0 + 1 < nch)
        def _():
            compute_chunk(c0 + 1, 1)

        return carry

    lax.fori_loop(0, (nch + 1) // 2, pair_body, 0)
    pltpu.sync_copy(out_v, out_hbm.at[pl.ds(wid * (nch * _CH), nch * _CH)])


def kernel(h_w_l, embedding):
    b = h_w_l.shape[0]
    n, d = embedding.shape
    bpw = b // _NW
    nch = bpw // _CH
    idx_all = h_w_l.reshape(_NW, nch, _CH, 3).transpose(0, 1, 3, 2)

    mesh = plsc.VectorSubcoreMesh(core_axis_name="c", subcore_axis_name="s")
    fn = pl.kernel(
        functools.partial(_sc_body, nch, d),
        out_type=jax.ShapeDtypeStruct((b,), jnp.float32),
        mesh=mesh,
        compiler_params=pltpu.CompilerParams(needs_layout_passes=False),
        scratch_types=[
            pltpu.VMEM((nch, 3, _CH), jnp.int32),
            pltpu.VMEM((2, 3, _CH, d), jnp.float32),
            pltpu.VMEM((bpw,), jnp.float32),
            pltpu.SemaphoreType.DMA((2,)),
        ],
    )
    return fn(idx_all, embedding)
